# baseline (speedup 1.0000x reference)
import functools

import jax
import jax.numpy as jnp
from jax import lax
from jax.experimental import pallas as pl
from jax.experimental.pallas import tpu as pltpu
from jax.experimental.pallas import tpu_sc as plsc

D = 512
V = 32128
S = 200
NRET = 4
T = 16
BSZ = 8
J = NRET * S
JP = 1024
ROWS = BSZ * T
NC = 2
NS = 16
NW = NC * NS
RPW = ROWS // NW


def _attn_body(hs_ref, embs_ref, ids_ref, logits_ref, wq_ref, bq_ref,
               wk_ref, bk_ref, wv_ref, bv_ref, wqp_ref, wcp_ref, bptr_ref,
               dense_ref, w_ref):
    hs = hs_ref[0]
    q = jnp.dot(hs, wq_ref[...], preferred_element_type=jnp.float32) + bq_ref[0]
    qk = lax.dot_general(q, wk_ref[...], (((1,), (1,)), ((), ())),
                         preferred_element_type=jnp.float32)
    qbk = jnp.sum(q * bk_ref[0], axis=-1, keepdims=True)
    hw = jnp.sum(hs * wqp_ref[0], axis=-1, keepdims=True)
    u_row = lax.dot_general(wcp_ref[...], wv_ref[...], (((1,), (1,)), ((), ())),
                            preferred_element_type=jnp.float32)
    ubv = jnp.sum(wcp_ref[0] * bv_ref[0])

    embs2 = embs_ref[0].reshape(NRET * S, D)
    scores = lax.dot_general(qk, embs2, (((1,), (1,)), ((), ())),
                             preferred_element_type=jnp.float32)
    scores = (scores + qbk) * jnp.float32(1.0 / (D ** 0.5))
    ids_row = ids_ref[0]
    scores = jnp.where(ids_row == 0, jnp.float32(-1e9), scores)

    seg = lax.broadcasted_iota(jnp.int32, (1, J), 1) // S
    segm = [seg == r for r in range(NRET)]
    m = jnp.max(scores, axis=-1, keepdims=True)
    e = jnp.exp(scores - m)
    sfull = jnp.zeros((T, 1), jnp.float32)
    srs = []
    for r in range(NRET):
        sr = jnp.sum(jnp.where(segm[r], e, 0.0), axis=-1, keepdims=True)
        srs.append(sr)
    sfull = jnp.where(segm[0], srs[0], 0.0)
    for r in range(1, NRET):
        sfull = jnp.where(segm[r], srs[r], sfull)
    probs = e / sfull

    embsu = lax.dot_general(u_row, embs2, (((1,), (1,)), ((), ())),
                            preferred_element_type=jnp.float32)
    pu = probs * embsu
    pcs = []
    for r in range(NRET):
        gr = jnp.sum(jnp.where(segm[r], pu, 0.0), axis=-1, keepdims=True)
        pcs.append(jax.nn.sigmoid(hw + gr + ubv + bptr_ref[0, 0]))
    pc_full = jnp.where(segm[0], pcs[0], 0.0)
    for r in range(1, NRET):
        pc_full = jnp.where(segm[r], pcs[r], pc_full)
    w_all = probs * pc_full * jnp.float32(1.0 / NRET)
    w_ref[0] = jnp.concatenate(
        [w_all, jnp.zeros((T, JP - J), jnp.float32)], axis=1)

    scale = 1.0 - (pcs[0] + pcs[1] + pcs[2] + pcs[3]) * jnp.float32(1.0 / NRET)
    l = logits_ref[0]
    ml = jnp.max(l, axis=-1, keepdims=True)
    e2 = jnp.exp(l - ml)
    s2 = jnp.sum(e2, axis=-1, keepdims=True)
    dense_val = e2 * (scale / s2)
    for t in range(T):
        dense_ref[pl.ds(t * V, V)] = dense_val[t]


@functools.cache
def _make_scatter_kernel(bh, boff):
    rows = bh * T
    rpw = rows // NW
    wpb = T // rpw
    mesh = plsc.VectorSubcoreMesh(
        core_axis_name="c", subcore_axis_name="s",
        num_cores=NC, num_subcores=NS)

    @functools.partial(
        pl.kernel,
        mesh=mesh,
        out_type=jax.ShapeDtypeStruct((bh, T, V), jnp.float32),
        scratch_types=[
            pltpu.VMEM((2, V + 8), jnp.float32),
            pltpu.VMEM((2, JP), jnp.float32),
            pltpu.VMEM((JP,), jnp.int32),
            pltpu.SemaphoreType.DMA((2,)),
            pltpu.SemaphoreType.DMA((2,)),
        ],
        compiler_params=pltpu.CompilerParams(
            needs_layout_passes=False, use_tc_tiling_on_sc=False),
    )
    def _scatter_kernel(dense_hbm, w_hbm, ids_hbm, out_hbm,
                        row_v, w_v, ids_v, in_sems, out_sems):
        RPW = rpw
        wid = lax.axis_index("s") * NC + lax.axis_index("c")
        b = wid // wpb
        t0 = (wid % wpb) * RPW
        pltpu.sync_copy(ids_hbm.at[pl.ds((b + boff) * JP, JP)], ids_v)

        def row_copies(i, sl):
            t = t0 + i
            row = b * T + t
            return (
                pltpu.make_async_copy(
                    dense_hbm.at[pl.ds(row * V, V)],
                    row_v.at[sl, pl.ds(0, V)], in_sems.at[sl]),
                pltpu.make_async_copy(
                    w_hbm.at[b, t], w_v.at[sl], in_sems.at[sl]),
            )

        def out_copy(i, sl):
            return pltpu.make_async_copy(
                row_v.at[sl, pl.ds(0, V)],
                out_hbm.at[b, t0 + i], out_sems.at[sl])

        for cp in row_copies(0, 0):
            cp.start()
        for i in range(RPW):
            sl = i % 2
            if i + 1 < RPW:
                if i >= 1:
                    out_copy(i - 1, 1 - sl).wait()
                for cp in row_copies(i + 1, 1 - sl):
                    cp.start()
            for cp in row_copies(i, sl):
                cp.wait()
            for jblk in range(JP // 16):
                idx = ids_v[pl.ds(jblk * 16, 16)]
                val = w_v[sl, pl.ds(jblk * 16, 16)]
                plsc.addupdate_scatter(row_v.at[sl], [idx], val)
            out_copy(i, sl).start()
        for i in range(RPW - 2, RPW):
            out_copy(i, i % 2).wait()

    return _scatter_kernel


def kernel(hidden_states, ret_text_embs, ret_input_ids, logits,
           Wq_attn, bq_attn, Wk_attn, bk_attn, Wv_attn, bv_attn,
           Wq_ptr, bq_ptr, Wc_ptr, bc_ptr):
    ids = ret_input_ids.astype(jnp.int32)
    ids3 = ids.reshape(BSZ, 1, J)
    bq2 = bq_attn.reshape(1, D)
    bk2 = bk_attn.reshape(1, D)
    bv2 = bv_attn.reshape(1, D)
    wqp = Wq_ptr.reshape(1, D)
    wcp = Wc_ptr.reshape(1, D)
    bptr = (bq_ptr + bc_ptr).reshape(1, 1)

    ids_sc = jnp.pad(ids.reshape(BSZ, J), ((0, 0), (0, JP - J)),
                     constant_values=V).reshape(BSZ * JP)

    bh = BSZ // 2
    halves = []
    for boff in (0, bh):
        dense, w = pl.pallas_call(
            _attn_body,
            grid=(bh,),
            in_specs=[
                pl.BlockSpec((1, T, D), lambda i, o=boff: (i + o, 0, 0)),
                pl.BlockSpec((1, NRET, S, D),
                             lambda i, o=boff: (i + o, 0, 0, 0)),
                pl.BlockSpec((1, 1, J), lambda i, o=boff: (i + o, 0, 0)),
                pl.BlockSpec((1, T, V), lambda i, o=boff: (i + o, 0, 0)),
                pl.BlockSpec((D, D), lambda i: (0, 0)),
                pl.BlockSpec((1, D), lambda i: (0, 0)),
                pl.BlockSpec((D, D), lambda i: (0, 0)),
                pl.BlockSpec((1, D), lambda i: (0, 0)),
                pl.BlockSpec((D, D), lambda i: (0, 0)),
                pl.BlockSpec((1, D), lambda i: (0, 0)),
                pl.BlockSpec((1, D), lambda i: (0, 0)),
                pl.BlockSpec((1, D), lambda i: (0, 0)),
                pl.BlockSpec((1, 1), lambda i: (0, 0)),
            ],
            out_specs=[
                pl.BlockSpec((T * V,), lambda i: (i,)),
                pl.BlockSpec((1, T, JP), lambda i: (i, 0, 0)),
            ],
            out_shape=[
                jax.ShapeDtypeStruct((bh * T * V,), jnp.float32),
                jax.ShapeDtypeStruct((bh, T, JP), jnp.float32),
            ],
        )(hidden_states, ret_text_embs, ids3, logits, Wq_attn, bq2,
          Wk_attn, bk2, Wv_attn, bv2, wqp, wcp, bptr)
        halves.append(_make_scatter_kernel(bh, boff)(dense, w, ids_sc))

    return jnp.concatenate(halves, axis=0)


# speedup vs baseline: 1.1341x; 1.1341x over previous
import functools

import jax
import jax.numpy as jnp
from jax import lax
from jax.experimental import pallas as pl
from jax.experimental.pallas import tpu as pltpu
from jax.experimental.pallas import tpu_sc as plsc

D = 512
V = 32128
S = 200
NRET = 4
T = 16
BSZ = 8
J = NRET * S
JP = 1024
ROWS = BSZ * T
NC = 2
NS = 16
NW = NC * NS
RPW = ROWS // NW


def _attn_body(hs_ref, embs_ref, ids_ref, logits_ref, wq_ref, bq_ref,
               wk_ref, bk_ref, wv_ref, bv_ref, wqp_ref, wcp_ref, bptr_ref,
               dense_ref, w_ref):
    hs = hs_ref[0]
    q = jnp.dot(hs, wq_ref[...], preferred_element_type=jnp.float32) + bq_ref[0]
    qk = lax.dot_general(q, wk_ref[...], (((1,), (1,)), ((), ())),
                         preferred_element_type=jnp.float32)
    qbk = jnp.sum(q * bk_ref[0], axis=-1, keepdims=True)
    hw = jnp.sum(hs * wqp_ref[0], axis=-1, keepdims=True)
    u_row = lax.dot_general(wcp_ref[...], wv_ref[...], (((1,), (1,)), ((), ())),
                            preferred_element_type=jnp.float32)
    ubv = jnp.sum(wcp_ref[0] * bv_ref[0])

    embs2 = embs_ref[0].reshape(NRET * S, D)
    scores = lax.dot_general(qk, embs2, (((1,), (1,)), ((), ())),
                             preferred_element_type=jnp.float32)
    scores = (scores + qbk) * jnp.float32(1.0 / (D ** 0.5))
    ids_row = ids_ref[0]
    scores = jnp.where(ids_row == 0, jnp.float32(-1e9), scores)

    seg = lax.broadcasted_iota(jnp.int32, (1, J), 1) // S
    segm = [seg == r for r in range(NRET)]
    m = jnp.max(scores, axis=-1, keepdims=True)
    e = jnp.exp(scores - m)
    sfull = jnp.zeros((T, 1), jnp.float32)
    srs = []
    for r in range(NRET):
        sr = jnp.sum(jnp.where(segm[r], e, 0.0), axis=-1, keepdims=True)
        srs.append(sr)
    sfull = jnp.where(segm[0], srs[0], 0.0)
    for r in range(1, NRET):
        sfull = jnp.where(segm[r], srs[r], sfull)
    probs = e / sfull

    embsu = lax.dot_general(u_row, embs2, (((1,), (1,)), ((), ())),
                            preferred_element_type=jnp.float32)
    pu = probs * embsu
    pcs = []
    for r in range(NRET):
        gr = jnp.sum(jnp.where(segm[r], pu, 0.0), axis=-1, keepdims=True)
        pcs.append(jax.nn.sigmoid(hw + gr + ubv + bptr_ref[0, 0]))
    pc_full = jnp.where(segm[0], pcs[0], 0.0)
    for r in range(1, NRET):
        pc_full = jnp.where(segm[r], pcs[r], pc_full)
    w_all = probs * pc_full * jnp.float32(1.0 / NRET)
    w_ref[0] = jnp.concatenate(
        [w_all, jnp.zeros((T, JP - J), jnp.float32)], axis=1)

    scale = 1.0 - (pcs[0] + pcs[1] + pcs[2] + pcs[3]) * jnp.float32(1.0 / NRET)
    l = logits_ref[0]
    ml = jnp.max(l, axis=-1, keepdims=True)
    e2 = jnp.exp(l - ml)
    s2 = jnp.sum(e2, axis=-1, keepdims=True)
    dense_val = e2 * (scale / s2)
    for t in range(T):
        dense_ref[pl.ds(t * V, V)] = dense_val[t]


@functools.cache
def _make_scatter_kernel(bh, boff):
    rows = bh * T
    rpw = rows // NW
    wpb = T // rpw
    mesh = plsc.VectorSubcoreMesh(
        core_axis_name="c", subcore_axis_name="s",
        num_cores=NC, num_subcores=NS)

    @functools.partial(
        pl.kernel,
        mesh=mesh,
        out_type=jax.ShapeDtypeStruct((bh, T, V), jnp.float32),
        scratch_types=[
            pltpu.VMEM((2, V + 8), jnp.float32),
            pltpu.VMEM((2, JP), jnp.float32),
            pltpu.VMEM((JP,), jnp.int32),
            pltpu.SemaphoreType.DMA((2,)),
            pltpu.SemaphoreType.DMA((2,)),
        ],
        compiler_params=pltpu.CompilerParams(
            needs_layout_passes=False, use_tc_tiling_on_sc=False),
    )
    def _scatter_kernel(dense_hbm, w_hbm, ids_hbm, out_hbm,
                        row_v, w_v, ids_v, in_sems, out_sems):
        RPW = rpw
        wid = lax.axis_index("s") * NC + lax.axis_index("c")
        b = wid // wpb
        t0 = (wid % wpb) * RPW
        pltpu.sync_copy(ids_hbm.at[pl.ds((b + boff) * JP, JP)], ids_v)

        def row_copies(i, sl):
            t = t0 + i
            row = b * T + t
            return (
                pltpu.make_async_copy(
                    dense_hbm.at[pl.ds(row * V, V)],
                    row_v.at[sl, pl.ds(0, V)], in_sems.at[sl]),
                pltpu.make_async_copy(
                    w_hbm.at[b, t], w_v.at[sl], in_sems.at[sl]),
            )

        def out_copy(i, sl):
            return pltpu.make_async_copy(
                row_v.at[sl, pl.ds(0, V)],
                out_hbm.at[b, t0 + i], out_sems.at[sl])

        for cp in row_copies(0, 0):
            cp.start()
        for i in range(RPW):
            sl = i % 2
            if i + 1 < RPW:
                if i >= 1:
                    out_copy(i - 1, 1 - sl).wait()
                for cp in row_copies(i + 1, 1 - sl):
                    cp.start()
            for cp in row_copies(i, sl):
                cp.wait()
            for jblk in range(JP // 16):
                idx = ids_v[pl.ds(jblk * 16, 16)]
                val = w_v[sl, pl.ds(jblk * 16, 16)]
                plsc.addupdate_scatter(row_v.at[sl], [idx], val)
            out_copy(i, sl).start()
        for i in range(RPW - 2, RPW):
            out_copy(i, i % 2).wait()

    return _scatter_kernel


def kernel(hidden_states, ret_text_embs, ret_input_ids, logits,
           Wq_attn, bq_attn, Wk_attn, bk_attn, Wv_attn, bv_attn,
           Wq_ptr, bq_ptr, Wc_ptr, bc_ptr):
    ids = ret_input_ids.astype(jnp.int32)
    ids3 = ids.reshape(BSZ, 1, J)
    bq2 = bq_attn.reshape(1, D)
    bk2 = bk_attn.reshape(1, D)
    bv2 = bv_attn.reshape(1, D)
    wqp = Wq_ptr.reshape(1, D)
    wcp = Wc_ptr.reshape(1, D)
    bptr = (bq_ptr + bc_ptr).reshape(1, 1)

    ids_sc = jnp.pad(ids.reshape(BSZ, J), ((0, 0), (0, JP - J)),
                     constant_values=V).reshape(BSZ * JP)

    bh = BSZ
    halves = []
    for boff in (0,):
        dense, w = pl.pallas_call(
            _attn_body,
            grid=(bh,),
            in_specs=[
                pl.BlockSpec((1, T, D), lambda i, o=boff: (i + o, 0, 0)),
                pl.BlockSpec((1, NRET, S, D),
                             lambda i, o=boff: (i + o, 0, 0, 0)),
                pl.BlockSpec((1, 1, J), lambda i, o=boff: (i + o, 0, 0)),
                pl.BlockSpec((1, T, V), lambda i, o=boff: (i + o, 0, 0)),
                pl.BlockSpec((D, D), lambda i: (0, 0)),
                pl.BlockSpec((1, D), lambda i: (0, 0)),
                pl.BlockSpec((D, D), lambda i: (0, 0)),
                pl.BlockSpec((1, D), lambda i: (0, 0)),
                pl.BlockSpec((D, D), lambda i: (0, 0)),
                pl.BlockSpec((1, D), lambda i: (0, 0)),
                pl.BlockSpec((1, D), lambda i: (0, 0)),
                pl.BlockSpec((1, D), lambda i: (0, 0)),
                pl.BlockSpec((1, 1), lambda i: (0, 0)),
            ],
            out_specs=[
                pl.BlockSpec((T * V,), lambda i: (i,)),
                pl.BlockSpec((1, T, JP), lambda i: (i, 0, 0)),
            ],
            out_shape=[
                jax.ShapeDtypeStruct((bh * T * V,), jnp.float32),
                jax.ShapeDtypeStruct((bh, T, JP), jnp.float32),
            ],
        )(hidden_states, ret_text_embs, ids3, logits, Wq_attn, bq2,
          Wk_attn, bk2, Wv_attn, bv2, wqp, wcp, bptr)
        halves.append(_make_scatter_kernel(bh, boff)(dense, w, ids_sc))

    return halves[0] if len(halves) == 1 else jnp.concatenate(halves, axis=0)


# speedup vs baseline: 1.1898x; 1.0491x over previous
import functools

import jax
import jax.numpy as jnp
from jax import lax
from jax.experimental import pallas as pl
from jax.experimental.pallas import tpu as pltpu
from jax.experimental.pallas import tpu_sc as plsc

D = 512
V = 32128
S = 200
NRET = 4
T = 16
BSZ = 8
J = NRET * S
JP = 1024
ROWS = BSZ * T
NC = 2
NS = 16
NW = NC * NS
RPW = ROWS // NW


def _attn_body(hs_ref, embs_ref, ids_ref, logits_ref, wq_ref, bq_ref,
               wk_ref, bk_ref, wv_ref, bv_ref, wqp_ref, wcp_ref, bptr_ref,
               dense_ref, w_ref, ids_out_ref):
    hs = hs_ref[0]
    q = jnp.dot(hs, wq_ref[...], preferred_element_type=jnp.float32) + bq_ref[0]
    qk = lax.dot_general(q, wk_ref[...], (((1,), (1,)), ((), ())),
                         preferred_element_type=jnp.float32)
    qbk = jnp.sum(q * bk_ref[0], axis=-1, keepdims=True)
    hw = jnp.sum(hs * wqp_ref[0], axis=-1, keepdims=True)
    u_row = lax.dot_general(wcp_ref[...], wv_ref[...], (((1,), (1,)), ((), ())),
                            preferred_element_type=jnp.float32)
    ubv = jnp.sum(wcp_ref[0] * bv_ref[0])

    embs2 = embs_ref[0].reshape(NRET * S, D)
    scores = lax.dot_general(qk, embs2, (((1,), (1,)), ((), ())),
                             preferred_element_type=jnp.float32)
    scores = (scores + qbk) * jnp.float32(1.0 / (D ** 0.5))
    ids_row = ids_ref[0]
    scores = jnp.where(ids_row == 0, jnp.float32(-1e9), scores)

    seg = lax.broadcasted_iota(jnp.int32, (1, J), 1) // S
    segm = [seg == r for r in range(NRET)]
    m = jnp.max(scores, axis=-1, keepdims=True)
    e = jnp.exp(scores - m)
    sfull = jnp.zeros((T, 1), jnp.float32)
    srs = []
    for r in range(NRET):
        sr = jnp.sum(jnp.where(segm[r], e, 0.0), axis=-1, keepdims=True)
        srs.append(sr)
    sfull = jnp.where(segm[0], srs[0], 0.0)
    for r in range(1, NRET):
        sfull = jnp.where(segm[r], srs[r], sfull)
    probs = e / sfull

    embsu = lax.dot_general(u_row, embs2, (((1,), (1,)), ((), ())),
                            preferred_element_type=jnp.float32)
    pu = probs * embsu
    pcs = []
    for r in range(NRET):
        gr = jnp.sum(jnp.where(segm[r], pu, 0.0), axis=-1, keepdims=True)
        pcs.append(jax.nn.sigmoid(hw + gr + ubv + bptr_ref[0, 0]))
    pc_full = jnp.where(segm[0], pcs[0], 0.0)
    for r in range(1, NRET):
        pc_full = jnp.where(segm[r], pcs[r], pc_full)
    w_all = jnp.concatenate(
        [probs * pc_full * jnp.float32(1.0 / NRET),
         jnp.zeros((T, JP - J), jnp.float32)], axis=1)
    for t in range(T):
        w_ref[pl.ds(t * JP, JP)] = w_all[t]
    ids_out_ref[...] = jnp.concatenate(
        [ids_row, jnp.full((1, JP - J), V, jnp.int32)], axis=1)[0]

    scale = 1.0 - (pcs[0] + pcs[1] + pcs[2] + pcs[3]) * jnp.float32(1.0 / NRET)
    l = logits_ref[0]
    ml = jnp.max(l, axis=-1, keepdims=True)
    e2 = jnp.exp(l - ml)
    s2 = jnp.sum(e2, axis=-1, keepdims=True)
    dense_val = e2 * (scale / s2)
    for t in range(T):
        dense_ref[pl.ds(t * V, V)] = dense_val[t]


@functools.cache
def _make_scatter_kernel(bh, boff):
    rows = bh * T
    rpw = rows // NW
    wpb = T // rpw
    mesh = plsc.VectorSubcoreMesh(
        core_axis_name="c", subcore_axis_name="s",
        num_cores=NC, num_subcores=NS)

    @functools.partial(
        pl.kernel,
        mesh=mesh,
        out_type=jax.ShapeDtypeStruct((bh, T, V), jnp.float32),
        scratch_types=[
            pltpu.VMEM((2, V + 8), jnp.float32),
            pltpu.VMEM((2, JP), jnp.float32),
            pltpu.VMEM((JP,), jnp.int32),
            pltpu.SemaphoreType.DMA((2,)),
            pltpu.SemaphoreType.DMA((2,)),
        ],
        compiler_params=pltpu.CompilerParams(
            needs_layout_passes=False, use_tc_tiling_on_sc=False),
    )
    def _scatter_kernel(dense_hbm, w_hbm, ids_hbm, out_hbm,
                        row_v, w_v, ids_v, in_sems, out_sems):
        RPW = rpw
        wid = lax.axis_index("s") * NC + lax.axis_index("c")
        b = wid // wpb
        t0 = (wid % wpb) * RPW
        pltpu.sync_copy(ids_hbm.at[pl.ds(b * JP, JP)], ids_v)

        def row_copies(i, sl):
            row = b * T + t0 + i
            return (
                pltpu.make_async_copy(
                    dense_hbm.at[pl.ds(row * V, V)],
                    row_v.at[sl, pl.ds(0, V)], in_sems.at[sl]),
                pltpu.make_async_copy(
                    w_hbm.at[pl.ds(row * JP, JP)], w_v.at[sl], in_sems.at[sl]),
            )

        def out_copy(i, sl):
            return pltpu.make_async_copy(
                row_v.at[sl, pl.ds(0, V)],
                out_hbm.at[b, t0 + i], out_sems.at[sl])

        for cp in row_copies(0, 0):
            cp.start()
        for i in range(RPW):
            sl = i % 2
            if i + 1 < RPW:
                if i >= 1:
                    out_copy(i - 1, 1 - sl).wait()
                for cp in row_copies(i + 1, 1 - sl):
                    cp.start()
            for cp in row_copies(i, sl):
                cp.wait()
            for jblk in range(JP // 16):
                idx = ids_v[pl.ds(jblk * 16, 16)]
                val = w_v[sl, pl.ds(jblk * 16, 16)]
                plsc.addupdate_scatter(row_v.at[sl], [idx], val)
            out_copy(i, sl).start()
        for i in range(RPW - 2, RPW):
            out_copy(i, i % 2).wait()

    return _scatter_kernel


def kernel(hidden_states, ret_text_embs, ret_input_ids, logits,
           Wq_attn, bq_attn, Wk_attn, bk_attn, Wv_attn, bv_attn,
           Wq_ptr, bq_ptr, Wc_ptr, bc_ptr):
    ids = ret_input_ids.astype(jnp.int32)
    ids3 = ids.reshape(BSZ, 1, J)
    bq2 = bq_attn.reshape(1, D)
    bk2 = bk_attn.reshape(1, D)
    bv2 = bv_attn.reshape(1, D)
    wqp = Wq_ptr.reshape(1, D)
    wcp = Wc_ptr.reshape(1, D)
    bptr = (bq_ptr + bc_ptr).reshape(1, 1)

    bh = BSZ
    halves = []
    for boff in (0,):
        outs = pl.pallas_call(
            _attn_body,
            grid=(bh,),
            in_specs=[
                pl.BlockSpec((1, T, D), lambda i, o=boff: (i + o, 0, 0)),
                pl.BlockSpec((1, NRET, S, D),
                             lambda i, o=boff: (i + o, 0, 0, 0)),
                pl.BlockSpec((1, 1, J), lambda i, o=boff: (i + o, 0, 0)),
                pl.BlockSpec((1, T, V), lambda i, o=boff: (i + o, 0, 0)),
                pl.BlockSpec((D, D), lambda i: (0, 0)),
                pl.BlockSpec((1, D), lambda i: (0, 0)),
                pl.BlockSpec((D, D), lambda i: (0, 0)),
                pl.BlockSpec((1, D), lambda i: (0, 0)),
                pl.BlockSpec((D, D), lambda i: (0, 0)),
                pl.BlockSpec((1, D), lambda i: (0, 0)),
                pl.BlockSpec((1, D), lambda i: (0, 0)),
                pl.BlockSpec((1, D), lambda i: (0, 0)),
                pl.BlockSpec((1, 1), lambda i: (0, 0)),
            ],
            out_specs=[
                pl.BlockSpec((T * V,), lambda i: (i,)),
                pl.BlockSpec((T * JP,), lambda i: (i,)),
                pl.BlockSpec((JP,), lambda i: (i,)),
            ],
            out_shape=[
                jax.ShapeDtypeStruct((bh * T * V,), jnp.float32),
                jax.ShapeDtypeStruct((bh * T * JP,), jnp.float32),
                jax.ShapeDtypeStruct((bh * JP,), jnp.int32),
            ],
        )(hidden_states, ret_text_embs, ids3, logits, Wq_attn, bq2,
          Wk_attn, bk2, Wv_attn, bv2, wqp, wcp, bptr)
        halves.append(_make_scatter_kernel(bh, boff)(*outs))

    return halves[0] if len(halves) == 1 else jnp.concatenate(halves, axis=0)


# speedup vs baseline: 1.1928x; 1.0025x over previous
import functools

import jax
import jax.numpy as jnp
from jax import lax
from jax.experimental import pallas as pl
from jax.experimental.pallas import tpu as pltpu
from jax.experimental.pallas import tpu_sc as plsc

D = 512
V = 32128
S = 200
NRET = 4
T = 16
BSZ = 8
J = NRET * S
JP = 1024
ROWS = BSZ * T
NC = 2
NS = 16
NW = NC * NS
RPW = ROWS // NW


def _attn_body(hs_ref, embs_ref, ids_ref, logits_ref, wq_ref, bq_ref,
               wk_ref, bk_ref, wv_ref, bv_ref, wqp_ref, wcp_ref, bptr_ref,
               dense_ref, w_ref, ids_out_ref):
    hs = hs_ref[0]
    q = jnp.dot(hs, wq_ref[...], preferred_element_type=jnp.float32) + bq_ref[0]
    qk = lax.dot_general(q, wk_ref[...], (((1,), (1,)), ((), ())),
                         preferred_element_type=jnp.float32)
    qbk = jnp.sum(q * bk_ref[0], axis=-1, keepdims=True)
    hw = jnp.sum(hs * wqp_ref[0], axis=-1, keepdims=True)
    u_row = lax.dot_general(wcp_ref[...], wv_ref[...], (((1,), (1,)), ((), ())),
                            preferred_element_type=jnp.float32)
    ubv = jnp.sum(wcp_ref[0] * bv_ref[0])

    embs2 = embs_ref[0].reshape(NRET * S, D)
    scores = lax.dot_general(qk, embs2, (((1,), (1,)), ((), ())),
                             preferred_element_type=jnp.float32)
    scores = (scores + qbk) * jnp.float32(1.0 / (D ** 0.5))
    ids_row = ids_ref[0]
    scores = jnp.where(ids_row == 0, jnp.float32(-1e9), scores)

    seg = lax.broadcasted_iota(jnp.int32, (1, J), 1) // S
    segm = [seg == r for r in range(NRET)]
    m = jnp.max(scores, axis=-1, keepdims=True)
    e = jnp.exp(scores - m)
    srs = []
    for r in range(NRET):
        sr = jnp.sum(jnp.where(segm[r], e, 0.0), axis=-1, keepdims=True)
        srs.append(sr)
    sfull = jnp.where(segm[0], srs[0], 0.0)
    for r in range(1, NRET):
        sfull = jnp.where(segm[r], srs[r], sfull)
    probs = e / sfull

    embsu = lax.dot_general(u_row, embs2, (((1,), (1,)), ((), ())),
                            preferred_element_type=jnp.float32)
    pu = probs * embsu
    pcs = []
    for r in range(NRET):
        gr = jnp.sum(jnp.where(segm[r], pu, 0.0), axis=-1, keepdims=True)
        pcs.append(jax.nn.sigmoid(hw + gr + ubv + bptr_ref[0, 0]))
    pc_full = jnp.where(segm[0], pcs[0], 0.0)
    for r in range(1, NRET):
        pc_full = jnp.where(segm[r], pcs[r], pc_full)
    w_all = jnp.concatenate(
        [probs * pc_full * jnp.float32(1.0 / NRET),
         jnp.zeros((T, JP - J), jnp.float32)], axis=1)
    for t in range(T):
        w_ref[pl.ds(t * JP, JP)] = w_all[t]
    ids_out_ref[...] = jnp.concatenate(
        [ids_row, jnp.full((1, JP - J), V, jnp.int32)], axis=1)[0]

    scale = 1.0 - (pcs[0] + pcs[1] + pcs[2] + pcs[3]) * jnp.float32(1.0 / NRET)
    l = logits_ref[0]
    ml = jnp.max(l, axis=-1, keepdims=True)
    e2 = jnp.exp(l - ml)
    s2 = jnp.sum(e2, axis=-1, keepdims=True)
    dense_val = e2 * (scale / s2)
    for t in range(T):
        dense_ref[pl.ds(t * V, V)] = dense_val[t]


@functools.cache
def _make_scatter_kernel(bh, boff):
    del boff
    rows = bh * T
    rpw = rows // NW
    wpb = T // rpw
    mesh = plsc.VectorSubcoreMesh(
        core_axis_name="c", subcore_axis_name="s",
        num_cores=NC, num_subcores=NS)

    @functools.partial(
        pl.kernel,
        mesh=mesh,
        out_type=jax.ShapeDtypeStruct((bh, T, V), jnp.float32),
        scratch_types=[
            pltpu.VMEM((2, V + 8), jnp.float32),
            pltpu.VMEM((2, JP), jnp.float32),
            pltpu.VMEM((JP,), jnp.int32),
            pltpu.SemaphoreType.DMA((2,)),
            pltpu.SemaphoreType.DMA((2,)),
        ],
        compiler_params=pltpu.CompilerParams(
            needs_layout_passes=False, use_tc_tiling_on_sc=False),
    )
    def _scatter_kernel(dense_hbm, w_hbm, ids_hbm, out_hbm,
                        row_v, w_v, ids_v, in_sems, out_sems):
        RPW = rpw
        wid = lax.axis_index("s") * NC + lax.axis_index("c")
        b = wid // wpb
        t0 = (wid % wpb) * RPW
        pltpu.sync_copy(ids_hbm.at[pl.ds(b * JP, JP)], ids_v)

        def row_copies(i, sl):
            row = b * T + t0 + i
            return (
                pltpu.make_async_copy(
                    dense_hbm.at[pl.ds(row * V, V)],
                    row_v.at[sl, pl.ds(0, V)], in_sems.at[sl]),
                pltpu.make_async_copy(
                    w_hbm.at[pl.ds(row * JP, JP)], w_v.at[sl], in_sems.at[sl]),
            )

        def out_copy(i, sl):
            return pltpu.make_async_copy(
                row_v.at[sl, pl.ds(0, V)],
                out_hbm.at[b, t0 + i], out_sems.at[sl])

        for cp in row_copies(0, 0):
            cp.start()
        for i in range(RPW):
            sl = i % 2
            if i + 1 < RPW:
                if i >= 1:
                    out_copy(i - 1, 1 - sl).wait()
                for cp in row_copies(i + 1, 1 - sl):
                    cp.start()
            for cp in row_copies(i, sl):
                cp.wait()
            for jblk in range(JP // 16):
                idx = ids_v[pl.ds(jblk * 16, 16)]
                val = w_v[sl, pl.ds(jblk * 16, 16)]
                plsc.addupdate_scatter(row_v.at[sl], [idx], val)
            out_copy(i, sl).start()
        for i in range(RPW - 2, RPW):
            out_copy(i, i % 2).wait()

    return _scatter_kernel


def kernel(hidden_states, ret_text_embs, ret_input_ids, logits,
           Wq_attn, bq_attn, Wk_attn, bk_attn, Wv_attn, bv_attn,
           Wq_ptr, bq_ptr, Wc_ptr, bc_ptr):
    ids = ret_input_ids.astype(jnp.int32)
    ids3 = ids.reshape(BSZ, 1, J)
    bq2 = bq_attn.reshape(1, D)
    bk2 = bk_attn.reshape(1, D)
    bv2 = bv_attn.reshape(1, D)
    wqp = Wq_ptr.reshape(1, D)
    wcp = Wc_ptr.reshape(1, D)
    bptr = (bq_ptr + bc_ptr).reshape(1, 1)

    bh = BSZ
    halves = []
    for boff in (0,):
        outs = pl.pallas_call(
            _attn_body,
            grid=(bh,),
            in_specs=[
                pl.BlockSpec((1, T, D), lambda i, o=boff: (i + o, 0, 0)),
                pl.BlockSpec((1, NRET, S, D),
                             lambda i, o=boff: (i + o, 0, 0, 0)),
                pl.BlockSpec((1, 1, J), lambda i, o=boff: (i + o, 0, 0)),
                pl.BlockSpec((1, T, V), lambda i, o=boff: (i + o, 0, 0)),
                pl.BlockSpec((D, D), lambda i: (0, 0)),
                pl.BlockSpec((1, D), lambda i: (0, 0)),
                pl.BlockSpec((D, D), lambda i: (0, 0)),
                pl.BlockSpec((1, D), lambda i: (0, 0)),
                pl.BlockSpec((D, D), lambda i: (0, 0)),
                pl.BlockSpec((1, D), lambda i: (0, 0)),
                pl.BlockSpec((1, D), lambda i: (0, 0)),
                pl.BlockSpec((1, D), lambda i: (0, 0)),
                pl.BlockSpec((1, 1), lambda i: (0, 0)),
            ],
            out_specs=[
                pl.BlockSpec((T * V,), lambda i: (i,)),
                pl.BlockSpec((T * JP,), lambda i: (i,)),
                pl.BlockSpec((JP,), lambda i: (i,)),
            ],
            out_shape=[
                jax.ShapeDtypeStruct((bh * T * V,), jnp.float32),
                jax.ShapeDtypeStruct((bh * T * JP,), jnp.float32),
                jax.ShapeDtypeStruct((bh * JP,), jnp.int32),
            ],
        )(hidden_states, ret_text_embs, ids3, logits, Wq_attn, bq2,
          Wk_attn, bk2, Wv_attn, bv2, wqp, wcp, bptr)
        halves.append(_make_scatter_kernel(bh, boff)(*outs))

    return halves[0] if len(halves) == 1 else jnp.concatenate(halves, axis=0)


# speedup vs baseline: 1.2193x; 1.0222x over previous
import functools

import jax
import jax.numpy as jnp
from jax import lax
from jax.experimental import pallas as pl
from jax.experimental.pallas import tpu as pltpu
from jax.experimental.pallas import tpu_sc as plsc

D = 512
V = 32128
S = 200
NRET = 4
T = 16
BSZ = 8
J = NRET * S
JP = 1024
ROWS = BSZ * T
NC = 2
NS = 16
NW = NC * NS
RPW = ROWS // NW


def _attn_body(hs_ref, embs_ref, ids_ref, logits_ref, wq_ref, bq_ref,
               wk_ref, bk_ref, wv_ref, bv_ref, wqp_ref, wcp_ref, bptr_ref,
               dense_ref, w_ref, ids_out_ref):
    hs = hs_ref[0]
    q = jnp.dot(hs, wq_ref[...], preferred_element_type=jnp.float32) + bq_ref[0]
    qk = lax.dot_general(q, wk_ref[...], (((1,), (1,)), ((), ())),
                         preferred_element_type=jnp.float32)
    qbk = jnp.sum(q * bk_ref[0], axis=-1, keepdims=True)
    hw = jnp.sum(hs * wqp_ref[0], axis=-1, keepdims=True)
    u_row = lax.dot_general(wcp_ref[...], wv_ref[...], (((1,), (1,)), ((), ())),
                            preferred_element_type=jnp.float32)
    ubv = jnp.sum(wcp_ref[0] * bv_ref[0])

    embs2 = embs_ref[0].reshape(NRET * S, D)
    scores = lax.dot_general(qk, embs2, (((1,), (1,)), ((), ())),
                             preferred_element_type=jnp.float32)
    scores = (scores + qbk) * jnp.float32(1.0 / (D ** 0.5))

    seg = lax.broadcasted_iota(jnp.int32, (1, J), 1) // S
    segm = [seg == r for r in range(NRET)]
    ids4 = ids_ref[0]
    i48 = jnp.concatenate([ids4] * NRET, axis=1)
    ids_row = jnp.where(segm[0], i48[0:1], 0)
    for r in range(1, NRET):
        ids_row = jnp.where(segm[r], i48[r:r + 1], ids_row)
    scores = jnp.where(ids_row == 0, jnp.float32(-1e9), scores)
    m = jnp.max(scores, axis=-1, keepdims=True)
    e = jnp.exp(scores - m)
    srs = []
    for r in range(NRET):
        sr = jnp.sum(jnp.where(segm[r], e, 0.0), axis=-1, keepdims=True)
        srs.append(sr)
    sfull = jnp.where(segm[0], srs[0], 0.0)
    for r in range(1, NRET):
        sfull = jnp.where(segm[r], srs[r], sfull)
    probs = e / sfull

    embsu = lax.dot_general(u_row, embs2, (((1,), (1,)), ((), ())),
                            preferred_element_type=jnp.float32)
    pu = probs * embsu
    pcs = []
    for r in range(NRET):
        gr = jnp.sum(jnp.where(segm[r], pu, 0.0), axis=-1, keepdims=True)
        pcs.append(jax.nn.sigmoid(hw + gr + ubv + bptr_ref[0, 0]))
    pc_full = jnp.where(segm[0], pcs[0], 0.0)
    for r in range(1, NRET):
        pc_full = jnp.where(segm[r], pcs[r], pc_full)
    w_all = jnp.concatenate(
        [probs * pc_full * jnp.float32(1.0 / NRET),
         jnp.zeros((T, JP - J), jnp.float32)], axis=1)
    for t in range(T):
        w_ref[pl.ds(t * JP, JP)] = w_all[t]
    ids_out_ref[...] = jnp.concatenate(
        [ids_row, jnp.full((1, JP - J), V, jnp.int32)], axis=1)[0]

    scale = 1.0 - (pcs[0] + pcs[1] + pcs[2] + pcs[3]) * jnp.float32(1.0 / NRET)
    l = logits_ref[0]
    ml = jnp.max(l, axis=-1, keepdims=True)
    e2 = jnp.exp(l - ml)
    s2 = jnp.sum(e2, axis=-1, keepdims=True)
    dense_val = e2 * (scale / s2)
    for t in range(T):
        dense_ref[pl.ds(t * V, V)] = dense_val[t]


@functools.cache
def _make_scatter_kernel(bh, boff):
    del boff
    rows = bh * T
    rpw = rows // NW
    wpb = T // rpw
    mesh = plsc.VectorSubcoreMesh(
        core_axis_name="c", subcore_axis_name="s",
        num_cores=NC, num_subcores=NS)

    @functools.partial(
        pl.kernel,
        mesh=mesh,
        out_type=jax.ShapeDtypeStruct((bh, T, V), jnp.float32),
        scratch_types=[
            pltpu.VMEM((2, V + 8), jnp.float32),
            pltpu.VMEM((2, JP), jnp.float32),
            pltpu.VMEM((JP,), jnp.int32),
            pltpu.SemaphoreType.DMA((2,)),
            pltpu.SemaphoreType.DMA((2,)),
        ],
        compiler_params=pltpu.CompilerParams(
            needs_layout_passes=False, use_tc_tiling_on_sc=False),
    )
    def _scatter_kernel(dense_hbm, w_hbm, ids_hbm, out_hbm,
                        row_v, w_v, ids_v, in_sems, out_sems):
        RPW = rpw
        wid = lax.axis_index("s") * NC + lax.axis_index("c")
        b = wid // wpb
        t0 = (wid % wpb) * RPW
        pltpu.sync_copy(ids_hbm.at[pl.ds(b * JP, JP)], ids_v)

        def row_copies(i, sl):
            row = b * T + t0 + i
            return (
                pltpu.make_async_copy(
                    dense_hbm.at[pl.ds(row * V, V)],
                    row_v.at[sl, pl.ds(0, V)], in_sems.at[sl]),
                pltpu.make_async_copy(
                    w_hbm.at[pl.ds(row * JP, JP)], w_v.at[sl], in_sems.at[sl]),
            )

        def out_copy(i, sl):
            return pltpu.make_async_copy(
                row_v.at[sl, pl.ds(0, V)],
                out_hbm.at[b, t0 + i], out_sems.at[sl])

        for cp in row_copies(0, 0):
            cp.start()
        for i in range(RPW):
            sl = i % 2
            if i + 1 < RPW:
                if i >= 1:
                    out_copy(i - 1, 1 - sl).wait()
                for cp in row_copies(i + 1, 1 - sl):
                    cp.start()
            for cp in row_copies(i, sl):
                cp.wait()
            for jblk in range(JP // 16):
                idx = ids_v[pl.ds(jblk * 16, 16)]
                val = w_v[sl, pl.ds(jblk * 16, 16)]
                plsc.addupdate_scatter(row_v.at[sl], [idx], val)
            out_copy(i, sl).start()
        for i in range(RPW - 2, RPW):
            out_copy(i, i % 2).wait()

    return _scatter_kernel


def kernel(hidden_states, ret_text_embs, ret_input_ids, logits,
           Wq_attn, bq_attn, Wk_attn, bk_attn, Wv_attn, bv_attn,
           Wq_ptr, bq_ptr, Wc_ptr, bc_ptr):
    ids = ret_input_ids.astype(jnp.int32)
    bq2 = bq_attn.reshape(1, D)
    bk2 = bk_attn.reshape(1, D)
    bv2 = bv_attn.reshape(1, D)
    wqp = Wq_ptr.reshape(1, D)
    wcp = Wc_ptr.reshape(1, D)
    bptr = (bq_ptr + bc_ptr).reshape(1, 1)

    bh = BSZ
    halves = []
    for boff in (0,):
        outs = pl.pallas_call(
            _attn_body,
            grid=(bh,),
            in_specs=[
                pl.BlockSpec((1, T, D), lambda i, o=boff: (i + o, 0, 0)),
                pl.BlockSpec((1, NRET, S, D),
                             lambda i, o=boff: (i + o, 0, 0, 0)),
                pl.BlockSpec((1, NRET, S), lambda i, o=boff: (i + o, 0, 0)),
                pl.BlockSpec((1, T, V), lambda i, o=boff: (i + o, 0, 0)),
                pl.BlockSpec((D, D), lambda i: (0, 0)),
                pl.BlockSpec((1, D), lambda i: (0, 0)),
                pl.BlockSpec((D, D), lambda i: (0, 0)),
                pl.BlockSpec((1, D), lambda i: (0, 0)),
                pl.BlockSpec((D, D), lambda i: (0, 0)),
                pl.BlockSpec((1, D), lambda i: (0, 0)),
                pl.BlockSpec((1, D), lambda i: (0, 0)),
                pl.BlockSpec((1, D), lambda i: (0, 0)),
                pl.BlockSpec((1, 1), lambda i: (0, 0)),
            ],
            out_specs=[
                pl.BlockSpec((T * V,), lambda i: (i,)),
                pl.BlockSpec((T * JP,), lambda i: (i,)),
                pl.BlockSpec((JP,), lambda i: (i,)),
            ],
            out_shape=[
                jax.ShapeDtypeStruct((bh * T * V,), jnp.float32),
                jax.ShapeDtypeStruct((bh * T * JP,), jnp.float32),
                jax.ShapeDtypeStruct((bh * JP,), jnp.int32),
            ],
        )(hidden_states, ret_text_embs, ids, logits, Wq_attn, bq2,
          Wk_attn, bk2, Wv_attn, bv2, wqp, wcp, bptr)
        halves.append(_make_scatter_kernel(bh, boff)(*outs))

    return halves[0] if len(halves) == 1 else jnp.concatenate(halves, axis=0)
